# Initial kernel scaffold; baseline (speedup 1.0000x reference)
import jax
import jax.numpy as jnp
from jax.experimental import pallas as pl


def kernel(shapes_list, colours_list, sym, shape_embed, colour_embed, W_ff, b_ff, W1, b1, W2, b2):
    raise NotImplementedError("write your pallas kernel here")


# speedup vs baseline: 62.7715x; 62.7715x over previous
import functools

import jax
import jax.numpy as jnp
from jax import lax
from jax.experimental import pallas as pl
from jax.experimental.pallas import tpu as pltpu
from jax.experimental.pallas import tpu_sc as plsc

B = 4096
L = 200
D = 64
NSHAPE = 26
NCOLOUR = 10
NCOMBO = NSHAPE * NCOLOUR

NC = 2
NS = 16
NW = NC * NS
ROWS_PER_W = B // NW
GROUPS = ROWS_PER_W // 16
BIN_WORDS = ROWS_PER_W * NCOMBO


TOK_WORDS = ROWS_PER_W * L


def _hist_body(shapes_hbm, colours_hbm, hist_hbm, sblk, cblk, bins):
    wid = lax.axis_index("s") * NC + lax.axis_index("c")
    tok0 = wid * TOK_WORDS

    pltpu.sync_copy(shapes_hbm.at[pl.ds(tok0, TOK_WORDS)], sblk)
    pltpu.sync_copy(colours_hbm.at[pl.ds(tok0, TOK_WORDS)], cblk)

    lane = jnp.arange(16, dtype=jnp.int32)
    zero16 = jnp.zeros((16,), jnp.float32)
    ones16 = jnp.ones((16,), jnp.float32)

    def zbody(i, carry):
        bins[pl.ds(i * 16, 16)] = zero16
        return carry

    lax.fori_loop(0, BIN_WORDS // 16, zbody, 0)

    for g in range(GROUPS):
        rowv = lane + (g * 16)
        rbase = rowv * NCOMBO
        tokbase = rowv * L

        def tbody(t, carry):
            tok = tokbase + t
            s = plsc.load_gather(sblk, [tok])
            c = plsc.load_gather(cblk, [tok])
            idx = s * NCOLOUR + c + rbase
            plsc.addupdate_scatter(bins, [idx], ones16)
            return carry

        lax.fori_loop(0, L, tbody, 0)

    pltpu.sync_copy(bins, hist_hbm.at[pl.ds(wid * BIN_WORDS, BIN_WORDS)])


def _histogram(shapes_i32, colours_i32):
    mesh = plsc.VectorSubcoreMesh(core_axis_name="c", subcore_axis_name="s")
    hist_flat = pl.kernel(
        _hist_body,
        mesh=mesh,
        compiler_params=pltpu.CompilerParams(needs_layout_passes=False),
        out_type=jax.ShapeDtypeStruct((B * NCOMBO,), jnp.float32),
        scratch_types=[
            pltpu.VMEM((TOK_WORDS,), jnp.int32),
            pltpu.VMEM((TOK_WORDS,), jnp.int32),
            pltpu.VMEM((BIN_WORDS,), jnp.float32),
        ],
    )(shapes_i32.reshape(-1), colours_i32.reshape(-1))
    return hist_flat.reshape(B, NCOMBO)


def _dense_body(hist_ref, srep_ref, ctile_ref, wff_ref, bff_ref, sym_ref,
                w1a_ref, w1b_ref, b1_ref, w2_ref, b2_ref, out_ref):
    e = srep_ref[...] + ctile_ref[...]
    t = jax.nn.relu(
        jnp.dot(e, wff_ref[...], preferred_element_type=jnp.float32)
        + bff_ref[...]
    )
    seq = jnp.dot(hist_ref[...], t, preferred_element_type=jnp.float32) * (1.0 / L)
    h = jax.nn.relu(
        jnp.dot(seq, w1a_ref[...], preferred_element_type=jnp.float32)
        + jnp.dot(sym_ref[...], w1b_ref[...], preferred_element_type=jnp.float32)
        + b1_ref[...]
    )
    out_ref[...] = (
        jnp.dot(h, w2_ref[...], preferred_element_type=jnp.float32) + b2_ref[...]
    )


def kernel(shapes_list, colours_list, sym, shape_embed, colour_embed,
           W_ff, b_ff, W1, b1, W2, b2):
    shapes_i32 = shapes_list.astype(jnp.int32)
    colours_i32 = colours_list.astype(jnp.int32)

    hist = _histogram(shapes_i32, colours_i32)

    s_rep = jnp.repeat(shape_embed, NCOLOUR, axis=0)
    c_tile = jnp.tile(colour_embed, (NSHAPE, 1))
    w1a = W1[:D, :]
    w1b = W1[D:, :]

    logits = pl.pallas_call(
        _dense_body,
        out_shape=jax.ShapeDtypeStruct((B, 2), jnp.float32),
    )(hist, s_rep, c_tile, W_ff, b_ff, sym, w1a, w1b, b1, W2, b2)
    return logits


# speedup vs baseline: 69.9516x; 1.1144x over previous
import functools

import jax
import jax.numpy as jnp
from jax import lax
from jax.experimental import pallas as pl
from jax.experimental.pallas import tpu as pltpu
from jax.experimental.pallas import tpu_sc as plsc

B = 4096
L = 200
D = 64
NSHAPE = 26
NCOLOUR = 10
NCOMBO = NSHAPE * NCOLOUR

NC = 2
NS = 16
NW = NC * NS
ROWS_PER_W = B // NW
GROUPS = ROWS_PER_W // 16
BIN_WORDS = ROWS_PER_W * NCOMBO


TOK_WORDS = ROWS_PER_W * L


TUNROLL = 8
ZUNROLL = 8


def _hist_body(shapes_hbm, colours_hbm, hist_hbm, sblk, cblk, bins, sem_s, sem_c):
    wid = lax.axis_index("s") * NC + lax.axis_index("c")
    tok0 = wid * TOK_WORDS

    cp_s = pltpu.async_copy(shapes_hbm.at[pl.ds(tok0, TOK_WORDS)], sblk, sem_s)
    cp_c = pltpu.async_copy(colours_hbm.at[pl.ds(tok0, TOK_WORDS)], cblk, sem_c)

    lane = jnp.arange(16, dtype=jnp.int32)
    zero16 = jnp.zeros((16,), jnp.float32)
    ones16 = jnp.ones((16,), jnp.float32)

    def zbody(i, carry):
        for u in range(ZUNROLL):
            bins[pl.ds((i * ZUNROLL + u) * 16, 16)] = zero16
        return carry

    lax.fori_loop(0, BIN_WORDS // 16 // ZUNROLL, zbody, 0)

    cp_s.wait()
    cp_c.wait()

    for g in range(GROUPS):
        rowv = lane + (g * 16)
        rbase = rowv * NCOMBO
        tokbase = rowv * L

        def tbody(i, tok):
            for u in range(TUNROLL):
                s = plsc.load_gather(sblk, [tok])
                c = plsc.load_gather(cblk, [tok])
                idx = s * NCOLOUR + c + rbase
                plsc.addupdate_scatter(bins, [idx], ones16)
                tok = tok + 1
            return tok

        lax.fori_loop(0, L // TUNROLL, tbody, tokbase)

    pltpu.sync_copy(bins, hist_hbm.at[pl.ds(wid * BIN_WORDS, BIN_WORDS)])


def _histogram(shapes_i32, colours_i32):
    mesh = plsc.VectorSubcoreMesh(core_axis_name="c", subcore_axis_name="s")
    hist_flat = pl.kernel(
        _hist_body,
        mesh=mesh,
        compiler_params=pltpu.CompilerParams(needs_layout_passes=False),
        out_type=jax.ShapeDtypeStruct((B * NCOMBO,), jnp.float32),
        scratch_types=[
            pltpu.VMEM((TOK_WORDS,), jnp.int32),
            pltpu.VMEM((TOK_WORDS,), jnp.int32),
            pltpu.VMEM((BIN_WORDS,), jnp.float32),
            pltpu.SemaphoreType.DMA,
            pltpu.SemaphoreType.DMA,
        ],
    )(shapes_i32.reshape(-1), colours_i32.reshape(-1))
    return hist_flat.reshape(B, NCOMBO)


def _dense_body(hist_ref, srep_ref, ctile_ref, wff_ref, bff_ref, sym_ref,
                w1a_ref, w1b_ref, b1_ref, w2_ref, b2_ref, out_ref):
    e = srep_ref[...] + ctile_ref[...]
    t = jax.nn.relu(
        jnp.dot(e, wff_ref[...], preferred_element_type=jnp.float32)
        + bff_ref[...]
    )
    seq = jnp.dot(hist_ref[...], t, preferred_element_type=jnp.float32) * (1.0 / L)
    h = jax.nn.relu(
        jnp.dot(seq, w1a_ref[...], preferred_element_type=jnp.float32)
        + jnp.dot(sym_ref[...], w1b_ref[...], preferred_element_type=jnp.float32)
        + b1_ref[...]
    )
    out_ref[...] = (
        jnp.dot(h, w2_ref[...], preferred_element_type=jnp.float32) + b2_ref[...]
    )


def kernel(shapes_list, colours_list, sym, shape_embed, colour_embed,
           W_ff, b_ff, W1, b1, W2, b2):
    shapes_i32 = shapes_list.astype(jnp.int32)
    colours_i32 = colours_list.astype(jnp.int32)

    hist = _histogram(shapes_i32, colours_i32)

    s_rep = jnp.repeat(shape_embed, NCOLOUR, axis=0)
    c_tile = jnp.tile(colour_embed, (NSHAPE, 1))
    w1a = W1[:D, :]
    w1b = W1[D:, :]

    logits = pl.pallas_call(
        _dense_body,
        out_shape=jax.ShapeDtypeStruct((B, 2), jnp.float32),
    )(hist, s_rep, c_tile, W_ff, b_ff, sym, w1a, w1b, b1, W2, b2)
    return logits


# speedup vs baseline: 77.6731x; 1.1104x over previous
import functools

import jax
import jax.numpy as jnp
from jax import lax
from jax.experimental import pallas as pl
from jax.experimental.pallas import tpu as pltpu
from jax.experimental.pallas import tpu_sc as plsc

B = 4096
L = 200
D = 64
NSHAPE = 26
NCOLOUR = 10
NCOMBO = NSHAPE * NCOLOUR

NC = 2
NS = 16
NW = NC * NS
ROWS_PER_W = B // NW
GROUPS = ROWS_PER_W // 16
BIN_WORDS = ROWS_PER_W * NCOMBO
TOK_WORDS = ROWS_PER_W * L

TUNROLL = 8
ZUNROLL = 8


def _hist_body(combo_hbm, hist_hbm, tblk, bins, sem_t):
    wid = lax.axis_index("s") * NC + lax.axis_index("c")
    tok0 = wid * TOK_WORDS

    cp_t = pltpu.async_copy(combo_hbm.at[pl.ds(tok0, TOK_WORDS)], tblk, sem_t)

    lane = jnp.arange(16, dtype=jnp.int32)
    zero16 = jnp.zeros((16,), jnp.float32)
    ones16 = jnp.ones((16,), jnp.float32)

    def zbody(i, carry):
        for u in range(ZUNROLL):
            bins[pl.ds((i * ZUNROLL + u) * 16, 16)] = zero16
        return carry

    lax.fori_loop(0, BIN_WORDS // 16 // ZUNROLL, zbody, 0)

    cp_t.wait()

    for g in range(GROUPS):
        rowv = lane + (g * 16)
        rbase = rowv * NCOMBO
        tokbase = rowv * L

        def tbody(i, tok):
            for u in range(TUNROLL):
                c = plsc.load_gather(tblk, [tok])
                plsc.addupdate_scatter(bins, [c + rbase], ones16)
                tok = tok + 1
            return tok

        lax.fori_loop(0, L // TUNROLL, tbody, tokbase)

    pltpu.sync_copy(bins, hist_hbm.at[pl.ds(wid * BIN_WORDS, BIN_WORDS)])


def _histogram(combo_flat):
    mesh = plsc.VectorSubcoreMesh(core_axis_name="c", subcore_axis_name="s")
    hist_flat = pl.kernel(
        _hist_body,
        mesh=mesh,
        compiler_params=pltpu.CompilerParams(needs_layout_passes=False),
        out_type=jax.ShapeDtypeStruct((B * NCOMBO,), jnp.float32),
        scratch_types=[
            pltpu.VMEM((TOK_WORDS,), jnp.int32),
            pltpu.VMEM((BIN_WORDS,), jnp.float32),
            pltpu.SemaphoreType.DMA,
        ],
    )(combo_flat)
    return hist_flat.reshape(B, NCOMBO)


BBLK = 512


def _dense_body(hist_ref, srep_ref, ctile_ref, wff_ref, bff_ref, sym_ref,
                w1a_ref, w1b_ref, b1_ref, w2_ref, b2_ref, out_ref):
    e = srep_ref[...] + ctile_ref[...]
    t = jax.nn.relu(
        jnp.dot(e, wff_ref[...], preferred_element_type=jnp.float32)
        + bff_ref[...]
    )
    seq = jnp.dot(hist_ref[...], t, preferred_element_type=jnp.float32) * (1.0 / L)
    h = jax.nn.relu(
        jnp.dot(seq, w1a_ref[...], preferred_element_type=jnp.float32)
        + jnp.dot(sym_ref[...], w1b_ref[...], preferred_element_type=jnp.float32)
        + b1_ref[...]
    )
    out_ref[...] = (
        jnp.dot(h, w2_ref[...], preferred_element_type=jnp.float32) + b2_ref[...]
    )


def kernel(shapes_list, colours_list, sym, shape_embed, colour_embed,
           W_ff, b_ff, W1, b1, W2, b2):
    combo_flat = (
        shapes_list.astype(jnp.int32) * NCOLOUR + colours_list.astype(jnp.int32)
    ).reshape(-1)

    hist = _histogram(combo_flat)

    s_rep = jnp.repeat(shape_embed, NCOLOUR, axis=0)
    c_tile = jnp.tile(colour_embed, (NSHAPE, 1))
    w1a = W1[:D, :]
    w1b = W1[D:, :]

    nblk = B // BBLK
    full = lambda shape: pl.BlockSpec(shape, lambda i: (0,) * len(shape))
    logits = pl.pallas_call(
        _dense_body,
        grid=(nblk,),
        in_specs=[
            pl.BlockSpec((BBLK, NCOMBO), lambda i: (i, 0)),
            full((NCOMBO, D)),
            full((NCOMBO, D)),
            full((D, D)),
            full((D,)),
            pl.BlockSpec((BBLK, 3), lambda i: (i, 0)),
            full((D, D)),
            full((3, D)),
            full((D,)),
            full((D, 2)),
            full((2,)),
        ],
        out_specs=pl.BlockSpec((BBLK, 2), lambda i: (i, 0)),
        out_shape=jax.ShapeDtypeStruct((B, 2), jnp.float32),
    )(hist, s_rep, c_tile, W_ff, b_ff, sym, w1a, w1b, b1, W2, b2)
    return logits
